# initial kernel scaffold (unmeasured)
import jax
import jax.numpy as jnp
from jax import lax
from jax.experimental import pallas as pl
from jax.experimental.pallas import tpu as pltpu


def kernel(
    x,
):
    def body(*refs):
        pass

    out_shape = jax.ShapeDtypeStruct(..., jnp.float32)
    return pl.pallas_call(body, out_shape=out_shape)(...)


# baseline (device time: 13394 ns/iter reference)
import jax
import jax.numpy as jnp
from jax import lax
from jax.experimental import pallas as pl
from jax.experimental.pallas import tpu as pltpu


def kernel(x):
    m, n = x.shape[-2], x.shape[-1]

    def body(x_ref, out_ref, recv1_ref, recv2_ref, send_sems, recv_sems):
        my_x = lax.axis_index("x")
        my_y = lax.axis_index("y")
        y_nbr = (my_x, 1 - my_y)
        x_nbr = (1 - my_x, my_y)

        barrier_sem = pltpu.get_barrier_semaphore()
        for nbr in (y_nbr, x_nbr):
            pl.semaphore_signal(
                barrier_sem, inc=1,
                device_id=nbr, device_id_type=pl.DeviceIdType.MESH,
            )
        pl.semaphore_wait(barrier_sem, 2)

        rdma1 = pltpu.make_async_remote_copy(
            src_ref=x_ref,
            dst_ref=recv1_ref,
            send_sem=send_sems.at[0],
            recv_sem=recv_sems.at[0],
            device_id=y_nbr,
            device_id_type=pl.DeviceIdType.MESH,
        )
        rdma1.start()
        rdma1.wait()
        out_ref[...] = x_ref[0, 0] + recv1_ref[0, 0]

        rdma2 = pltpu.make_async_remote_copy(
            src_ref=out_ref,
            dst_ref=recv2_ref,
            send_sem=send_sems.at[1],
            recv_sem=recv_sems.at[1],
            device_id=x_nbr,
            device_id_type=pl.DeviceIdType.MESH,
        )
        rdma2.start()
        rdma2.wait()
        out_ref[...] = out_ref[...] + recv2_ref[...]

    return pl.pallas_call(
        body,
        out_shape=jax.ShapeDtypeStruct((m, n), jnp.float32),
        in_specs=[pl.BlockSpec(memory_space=pltpu.VMEM)],
        out_specs=pl.BlockSpec(memory_space=pltpu.VMEM),
        scratch_shapes=[
            pltpu.VMEM((1, 1, m, n), jnp.float32),
            pltpu.VMEM((m, n), jnp.float32),
            pltpu.SemaphoreType.DMA((2,)),
            pltpu.SemaphoreType.DMA((2,)),
        ],
        compiler_params=pltpu.CompilerParams(collective_id=0),
    )(x)


# device time: 10597 ns/iter; 1.2639x vs baseline; 1.2639x over previous
import jax
import jax.numpy as jnp
from jax import lax
from jax.experimental import pallas as pl
from jax.experimental.pallas import tpu as pltpu


def kernel(x):
    m, n = x.shape[-2], x.shape[-1]
    h = m // 2
    x = x.reshape(m, n)

    def body(x_ref, out_ref, ra1, rb1, ra2, rb2, send_sems, recv_sems):
        my_x = lax.axis_index("x")
        my_y = lax.axis_index("y")
        y_nbr = (my_x, 1 - my_y)
        x_nbr = (1 - my_x, my_y)

        barrier_sem = pltpu.get_barrier_semaphore()
        for nbr in (y_nbr, x_nbr):
            pl.semaphore_signal(
                barrier_sem, inc=1,
                device_id=nbr, device_id_type=pl.DeviceIdType.MESH,
            )
        pl.semaphore_wait(barrier_sem, 2)

        rdma_a1 = pltpu.make_async_remote_copy(
            src_ref=x_ref.at[pl.ds(0, h)],
            dst_ref=ra1,
            send_sem=send_sems.at[0],
            recv_sem=recv_sems.at[0],
            device_id=y_nbr,
            device_id_type=pl.DeviceIdType.MESH,
        )
        rdma_b1 = pltpu.make_async_remote_copy(
            src_ref=x_ref.at[pl.ds(h, h)],
            dst_ref=rb1,
            send_sem=send_sems.at[1],
            recv_sem=recv_sems.at[1],
            device_id=x_nbr,
            device_id_type=pl.DeviceIdType.MESH,
        )
        rdma_a1.start()
        rdma_b1.start()

        rdma_a1.wait_recv()
        out_ref[pl.ds(0, h), :] = x_ref[pl.ds(0, h), :] + ra1[...]
        rdma_a2 = pltpu.make_async_remote_copy(
            src_ref=out_ref.at[pl.ds(0, h)],
            dst_ref=ra2,
            send_sem=send_sems.at[2],
            recv_sem=recv_sems.at[2],
            device_id=x_nbr,
            device_id_type=pl.DeviceIdType.MESH,
        )
        rdma_a2.start()

        rdma_b1.wait_recv()
        out_ref[pl.ds(h, h), :] = x_ref[pl.ds(h, h), :] + rb1[...]
        rdma_b2 = pltpu.make_async_remote_copy(
            src_ref=out_ref.at[pl.ds(h, h)],
            dst_ref=rb2,
            send_sem=send_sems.at[3],
            recv_sem=recv_sems.at[3],
            device_id=y_nbr,
            device_id_type=pl.DeviceIdType.MESH,
        )
        rdma_b2.start()

        rdma_a2.wait()
        out_ref[pl.ds(0, h), :] = out_ref[pl.ds(0, h), :] + ra2[...]
        rdma_b2.wait()
        out_ref[pl.ds(h, h), :] = out_ref[pl.ds(h, h), :] + rb2[...]

        rdma_a1.wait_send()
        rdma_b1.wait_send()

    return pl.pallas_call(
        body,
        out_shape=jax.ShapeDtypeStruct((m, n), jnp.float32),
        in_specs=[pl.BlockSpec(memory_space=pltpu.VMEM)],
        out_specs=pl.BlockSpec(memory_space=pltpu.VMEM),
        scratch_shapes=[
            pltpu.VMEM((h, n), jnp.float32),
            pltpu.VMEM((h, n), jnp.float32),
            pltpu.VMEM((h, n), jnp.float32),
            pltpu.VMEM((h, n), jnp.float32),
            pltpu.SemaphoreType.DMA((4,)),
            pltpu.SemaphoreType.DMA((4,)),
        ],
        compiler_params=pltpu.CompilerParams(collective_id=0),
    )(x)


# device time: 9984 ns/iter; 1.3415x vs baseline; 1.0614x over previous
import jax
import jax.numpy as jnp
from jax import lax
from jax.experimental import pallas as pl
from jax.experimental.pallas import tpu as pltpu

N_CHUNK = 2


def kernel(x):
    m, n = x.shape[-2], x.shape[-1]
    h = m // 2
    c = h // N_CHUNK
    x = x.reshape(m, n)

    def body(x_ref, out_ref, r1, r2, send_sems, recv_sems):
        my_x = lax.axis_index("x")
        my_y = lax.axis_index("y")
        y_nbr = (my_x, 1 - my_y)
        x_nbr = (1 - my_x, my_y)

        chunks = []
        for j in range(N_CHUNK):
            chunks.append((j, j * c, y_nbr, x_nbr))
            chunks.append((N_CHUNK + j, h + j * c, x_nbr, y_nbr))

        barrier_sem = pltpu.get_barrier_semaphore()
        for nbr in (y_nbr, x_nbr):
            pl.semaphore_signal(
                barrier_sem, inc=1,
                device_id=nbr, device_id_type=pl.DeviceIdType.MESH,
            )
        pl.semaphore_wait(barrier_sem, 2)

        p1 = []
        for i, off, peer1, _ in chunks:
            r = pltpu.make_async_remote_copy(
                src_ref=x_ref.at[pl.ds(off, c)],
                dst_ref=r1.at[pl.ds(off, c)],
                send_sem=send_sems.at[i],
                recv_sem=recv_sems.at[i],
                device_id=peer1,
                device_id_type=pl.DeviceIdType.MESH,
            )
            r.start()
            p1.append(r)

        p2 = []
        for k, (i, off, _, peer2) in enumerate(chunks):
            p1[k].wait_recv()
            out_ref[pl.ds(off, c), :] = (
                x_ref[pl.ds(off, c), :] + r1[pl.ds(off, c), :]
            )
            r = pltpu.make_async_remote_copy(
                src_ref=out_ref.at[pl.ds(off, c)],
                dst_ref=r2.at[pl.ds(off, c)],
                send_sem=send_sems.at[2 * N_CHUNK + i],
                recv_sem=recv_sems.at[2 * N_CHUNK + i],
                device_id=peer2,
                device_id_type=pl.DeviceIdType.MESH,
            )
            r.start()
            p2.append(r)

        for k, (i, off, _, _) in enumerate(chunks):
            p2[k].wait()
            out_ref[pl.ds(off, c), :] = (
                out_ref[pl.ds(off, c), :] + r2[pl.ds(off, c), :]
            )

        for r in p1:
            r.wait_send()

    return pl.pallas_call(
        body,
        out_shape=jax.ShapeDtypeStruct((m, n), jnp.float32),
        in_specs=[pl.BlockSpec(memory_space=pltpu.VMEM)],
        out_specs=pl.BlockSpec(memory_space=pltpu.VMEM),
        scratch_shapes=[
            pltpu.VMEM((m, n), jnp.float32),
            pltpu.VMEM((m, n), jnp.float32),
            pltpu.SemaphoreType.DMA((4 * N_CHUNK,)),
            pltpu.SemaphoreType.DMA((4 * N_CHUNK,)),
        ],
        compiler_params=pltpu.CompilerParams(collective_id=0),
    )(x)


# device time: 9787 ns/iter; 1.3686x vs baseline; 1.0201x over previous
import jax
import jax.numpy as jnp
from jax import lax
from jax.experimental import pallas as pl
from jax.experimental.pallas import tpu as pltpu

N_CHUNK = 4


def kernel(x):
    m, n = x.shape[-2], x.shape[-1]
    h = m // 2
    c = h // N_CHUNK
    x = x.reshape(m, n)

    def body(x_ref, out_ref, r1, r2, send_sems, recv_sems):
        my_x = lax.axis_index("x")
        my_y = lax.axis_index("y")
        y_nbr = (my_x, 1 - my_y)
        x_nbr = (1 - my_x, my_y)

        chunks = []
        for j in range(N_CHUNK):
            chunks.append((j, j * c, y_nbr, x_nbr))
            chunks.append((N_CHUNK + j, h + j * c, x_nbr, y_nbr))

        barrier_sem = pltpu.get_barrier_semaphore()
        for nbr in (y_nbr, x_nbr):
            pl.semaphore_signal(
                barrier_sem, inc=1,
                device_id=nbr, device_id_type=pl.DeviceIdType.MESH,
            )
        pl.semaphore_wait(barrier_sem, 2)

        p1 = []
        for i, off, peer1, _ in chunks:
            r = pltpu.make_async_remote_copy(
                src_ref=x_ref.at[pl.ds(off, c)],
                dst_ref=r1.at[pl.ds(off, c)],
                send_sem=send_sems.at[i],
                recv_sem=recv_sems.at[i],
                device_id=peer1,
                device_id_type=pl.DeviceIdType.MESH,
            )
            r.start()
            p1.append(r)

        p2 = []
        for k, (i, off, _, peer2) in enumerate(chunks):
            p1[k].wait_recv()
            out_ref[pl.ds(off, c), :] = (
                x_ref[pl.ds(off, c), :] + r1[pl.ds(off, c), :]
            )
            r = pltpu.make_async_remote_copy(
                src_ref=out_ref.at[pl.ds(off, c)],
                dst_ref=r2.at[pl.ds(off, c)],
                send_sem=send_sems.at[2 * N_CHUNK + i],
                recv_sem=recv_sems.at[2 * N_CHUNK + i],
                device_id=peer2,
                device_id_type=pl.DeviceIdType.MESH,
            )
            r.start()
            p2.append(r)

        for k, (i, off, _, _) in enumerate(chunks):
            p2[k].wait()
            out_ref[pl.ds(off, c), :] = (
                out_ref[pl.ds(off, c), :] + r2[pl.ds(off, c), :]
            )

        for r in p1:
            r.wait_send()

    return pl.pallas_call(
        body,
        out_shape=jax.ShapeDtypeStruct((m, n), jnp.float32),
        in_specs=[pl.BlockSpec(memory_space=pltpu.VMEM)],
        out_specs=pl.BlockSpec(memory_space=pltpu.VMEM),
        scratch_shapes=[
            pltpu.VMEM((m, n), jnp.float32),
            pltpu.VMEM((m, n), jnp.float32),
            pltpu.SemaphoreType.DMA((4 * N_CHUNK,)),
            pltpu.SemaphoreType.DMA((4 * N_CHUNK,)),
        ],
        compiler_params=pltpu.CompilerParams(collective_id=0),
    )(x)
